# baseline (speedup 1.0000x reference)
import functools

import jax
import jax.numpy as jnp
from jax import lax
from jax.experimental import pallas as pl
from jax.experimental.pallas import tpu as pltpu, tpu_sc as plsc

_GAMMA = 12.0
_HID = 64
_BATCH = 4096
_NC = 2
_NS = 16
_NW = _NC * _NS
_BPW = _BATCH // _NW
_LANES = 16
_NBLK = _BPW // _LANES


def _sc_kernel(h_idx, r_idx, t_idx, ent, rel, out,
               hidx_v, ridx_v, tidx_v, h_rows, r_rows, t_rows, scores_v,
               sem):
    wid = lax.axis_index("s") * _NC + lax.axis_index("c")
    base = wid * _BPW

    pltpu.sync_copy(h_idx.at[pl.ds(base, _BPW)], hidx_v)
    pltpu.sync_copy(r_idx.at[pl.ds(base, _BPW)], ridx_v)
    pltpu.sync_copy(t_idx.at[pl.ds(base, _BPW)], tidx_v)
    c1 = pltpu.async_copy(ent.at[hidx_v], h_rows, sem)
    c2 = pltpu.async_copy(rel.at[ridx_v], r_rows, sem)
    c3 = pltpu.async_copy(ent.at[tidx_v], t_rows, sem)
    c1.wait()
    c2.wait()
    c3.wait()

    def block(blk, carry):
        row16 = lax.iota(jnp.int32, _LANES) + blk * _LANES
        acc = jnp.zeros((_LANES,), jnp.float32)
        for j in range(_HID):
            col16 = jnp.full((_LANES,), j, jnp.int32)
            hv = plsc.load_gather(h_rows, [row16, col16])
            rv = plsc.load_gather(r_rows, [row16, col16])
            tv = plsc.load_gather(t_rows, [row16, col16])
            acc = acc + jnp.abs(hv + rv - tv)
        scores_v[pl.ds(blk * _LANES, _LANES)] = _GAMMA - acc
        return carry

    lax.fori_loop(0, _NBLK, block, 0, unroll=False)

    pltpu.sync_copy(scores_v, out.at[pl.ds(base, _BPW)])


@jax.jit
def _score(h_idx, r_idx, t_idx, ent, rel):
    mesh = plsc.VectorSubcoreMesh(core_axis_name="c", subcore_axis_name="s")
    kfn = functools.partial(
        pl.kernel,
        mesh=mesh,
        compiler_params=pltpu.CompilerParams(
            needs_layout_passes=False, use_tc_tiling_on_sc=False),
        out_type=jax.ShapeDtypeStruct((_BATCH,), jnp.float32),
        scratch_types=[
            pltpu.VMEM((_BPW,), jnp.int32),
            pltpu.VMEM((_BPW,), jnp.int32),
            pltpu.VMEM((_BPW,), jnp.int32),
            pltpu.VMEM((_BPW, _HID), jnp.float32),
            pltpu.VMEM((_BPW, _HID), jnp.float32),
            pltpu.VMEM((_BPW, _HID), jnp.float32),
            pltpu.VMEM((_BPW,), jnp.float32),
            pltpu.SemaphoreType.DMA,
        ],
    )(_sc_kernel)
    return kfn(h_idx, r_idx, t_idx, ent, rel)


def kernel(sample, entity_embedding, relation_embedding):
    h_idx = sample[:, 0]
    r_idx = sample[:, 1]
    t_idx = sample[:, 2]
    score = _score(h_idx, r_idx, t_idx, entity_embedding, relation_embedding)
    return score[:, None]


# speedup vs baseline: 2.2351x; 2.2351x over previous
import functools

import jax
import jax.numpy as jnp
from jax import lax
from jax.experimental import pallas as pl
from jax.experimental.pallas import tpu as pltpu, tpu_sc as plsc

_GAMMA = 12.0
_HID = 64
_BATCH = 4096
_NLK = 3 * _BATCH
_NENT = 1000000
_NC = 2
_NS = 16
_NW = _NC * _NS
_LANES = 16
_TCOLS = 7813
_TPW = 245
_CPW = _TPW
_DUMP = _NLK
_GROWS = 12416
_CAP = 128
_SENT = 0x7FFFFFFF


def _sc_gather(lk, ent_t, rel_t, g_out,
               lk_v, lcol, ldst, cc, cd, buf, stag, stag_rows, dstage,
               cnt_s, sem_in):
    wid = lax.axis_index("s") * _NC + lax.axis_index("c")
    wt0 = wid * _TPW
    lo = wt0 * 128
    hi = lo + _TPW * 128

    cnt_s[0] = 0
    cnt_s[1] = 0
    iota = lax.iota(jnp.int32, _LANES)
    dump_vec = jnp.full((_LANES,), _DUMP, jnp.int32)
    for z in range(_CAP // _LANES):
        dstage[pl.ds(z * _LANES, _LANES)] = dump_vec

    pltpu.sync_copy(lk, lk_v)

    def scan_block(i, carry):
        v = lk_v[pl.ds(i * _LANES, _LANES)]
        m = (v >= lo) & (v < hi)
        n = cnt_s[0]
        plsc.store_compressed(lcol.at[pl.ds(n, _LANES)], v, mask=m)
        plsc.store_compressed(
            ldst.at[pl.ds(n, _LANES)], iota + i * _LANES, mask=m)
        cnt_s[0] = n + jnp.sum(jnp.where(m, 1, 0))
        return carry

    lax.fori_loop(0, _NLK // _LANES, scan_block, 0, unroll=False)
    n_total = cnt_s[0]
    lcol[pl.ds(n_total, _LANES)] = jnp.full((_LANES,), _SENT, jnp.int32)

    def fire(k, par):
        ch = wt0 + k

        @pl.when(ch < _TCOLS)
        def _():
            off = pl.multiple_of(ch * 128, 128)
            pltpu.async_copy(
                ent_t.at[:, pl.ds(off, 128)],
                buf.at[par, pl.ds(0, _HID)], sem_in)
            pltpu.async_copy(
                rel_t.at[:, pl.ds(off, 128)],
                buf.at[par, pl.ds(_HID, _HID)], sem_in)

    def wait(k, par):
        ch = wt0 + k

        @pl.when(ch < _TCOLS)
        def _():
            off = pl.multiple_of(ch * 128, 128)
            pltpu.make_async_copy(
                ent_t.at[:, pl.ds(off, 128)],
                buf.at[par, pl.ds(0, _HID)], sem_in).wait()
            pltpu.make_async_copy(
                rel_t.at[:, pl.ds(off, 128)],
                buf.at[par, pl.ds(_HID, _HID)], sem_in).wait()

    def flush():
        pltpu.sync_copy(stag_rows, g_out.at[dstage])
        for z in range(_CAP // _LANES):
            dstage[pl.ds(z * _LANES, _LANES)] = dump_vec

    def process(k, par):
        ch = wt0 + k

        @pl.when(ch < _TCOLS)
        def _():
            off = ch * 128
            cnt_s[2] = 0

            def rescan(q, carry):
                lc = lcol[pl.ds(q * _LANES, _LANES)]
                m = (lc >= off) & (lc < off + 128)
                mc = cnt_s[2]
                plsc.store_compressed(
                    cc.at[pl.ds(mc, _LANES)], lc - off, mask=m)
                plsc.store_compressed(
                    cd.at[pl.ds(mc, _LANES)],
                    ldst[pl.ds(q * _LANES, _LANES)], mask=m)
                cnt_s[2] = mc + jnp.sum(jnp.where(m, 1, 0))
                return carry

            nb = (n_total + _LANES - 1) // _LANES
            lax.fori_loop(0, nb, rescan, 0, unroll=False)
            mc = cnt_s[2]
            cc[pl.ds(mc, _LANES)] = jnp.zeros((_LANES,), jnp.int32)
            cd[pl.ds(mc, _LANES)] = dump_vec

            def extract(e, carry):
                j16 = cc[pl.ds(e * _LANES, _LANES)]
                d16 = cd[pl.ds(e * _LANES, _LANES)]
                rbase = jnp.where(
                    (d16 >= _BATCH) & (d16 < 2 * _BATCH), _HID, 0)
                for c in range(_HID):
                    stag[c, :] = plsc.load_gather(
                        buf.at[par], [rbase + c, j16])
                f = cnt_s[1]
                for q in range(_LANES):
                    colq = jnp.full((_LANES,), q, jnp.int32)
                    for a in range(_HID // _LANES):
                        t = plsc.load_gather(
                            stag, [a * _LANES + iota, colq])
                        stag_rows[f + q, pl.ds(a * _LANES, _LANES)] = t
                dstage[pl.ds(f, _LANES)] = d16
                f2 = f + jnp.minimum(mc - e * _LANES, _LANES)

                @pl.when(f2 >= _CAP - _LANES)
                def _():
                    flush()

                cnt_s[1] = jnp.where(f2 >= _CAP - _LANES, 0, f2)
                return carry

            ne = (mc + _LANES - 1) // _LANES
            lax.fori_loop(0, ne, extract, 0, unroll=False)

    fire(0, 0)

    def pair(j, carry):
        k0 = 2 * j
        wait(k0, 0)
        fire(k0 + 1, 1)
        process(k0, 0)
        wait(k0 + 1, 1)
        fire(k0 + 2, 0)
        process(k0 + 1, 1)
        return carry

    lax.fori_loop(0, (_CPW - 1) // 2, pair, 0, unroll=False)
    wait(_CPW - 1, 0)
    process(_CPW - 1, 0)
    flush()


def _tc_score(h_ref, r_ref, t_ref, o_ref):
    d = h_ref[:, :_HID] + r_ref[:, :_HID] - t_ref[:, :_HID]
    o_ref[...] = _GAMMA - jnp.sum(jnp.abs(d), axis=1, keepdims=True)


@jax.jit
def _score(lk, ent_t, rel_t):
    mesh = plsc.VectorSubcoreMesh(core_axis_name="c", subcore_axis_name="s")
    gather_fn = functools.partial(
        pl.kernel,
        mesh=mesh,
        compiler_params=pltpu.CompilerParams(
            needs_layout_passes=False, disable_bounds_checks=True),
        out_type=jax.ShapeDtypeStruct((_GROWS, 128), jnp.float32),
        scratch_types=[
            pltpu.VMEM((_NLK,), jnp.int32),
            pltpu.VMEM((_NLK + _LANES,), jnp.int32),
            pltpu.VMEM((_NLK + _LANES,), jnp.int32),
            pltpu.VMEM((_NLK + _LANES,), jnp.int32),
            pltpu.VMEM((_NLK + _LANES,), jnp.int32),
            pltpu.VMEM((2, 2 * _HID, 128), jnp.float32),
            pltpu.VMEM((_HID, _LANES), jnp.float32),
            pltpu.VMEM((_CAP, 128), jnp.float32),
            pltpu.VMEM((_CAP,), jnp.int32),
            pltpu.SMEM((4,), jnp.int32),
            pltpu.SemaphoreType.DMA,
        ],
    )(_sc_gather)
    g = gather_fn(lk, ent_t, rel_t)

    nblk = 8
    rows = _BATCH // nblk
    score = pl.pallas_call(
        _tc_score,
        grid=(nblk,),
        in_specs=[
            pl.BlockSpec((rows, 128), lambda i: (i, 0)),
            pl.BlockSpec((rows, 128), lambda i: (i + nblk, 0)),
            pl.BlockSpec((rows, 128), lambda i: (i + 2 * nblk, 0)),
        ],
        out_specs=pl.BlockSpec((rows, 1), lambda i: (i, 0)),
        out_shape=jax.ShapeDtypeStruct((_BATCH, 1), jnp.float32),
    )(g, g, g)
    return score


def kernel(sample, entity_embedding, relation_embedding):
    lk = jnp.concatenate([sample[:, 0], sample[:, 1], sample[:, 2]])
    return _score(lk, entity_embedding.T, relation_embedding.T)
